# Initial kernel scaffold; baseline (speedup 1.0000x reference)
import jax
import jax.numpy as jnp
from jax.experimental import pallas as pl


def kernel(x, edge_index, params):
    raise NotImplementedError("write your pallas kernel here")


# speedup vs baseline: 8.1879x; 8.1879x over previous
import functools

import jax
import jax.numpy as jnp
from jax import lax
from jax.experimental import pallas as pl
from jax.experimental.pallas import tpu as pltpu
from jax.experimental.pallas import tpu_sc as plsc

N = 10000
H = 128
E = 320000
EPS = 1e-5
NC = 2
NS = 16
NW = NC * NS
CH = 128
NSTEP = -(-E // (NW * CH))
EPAD = NW * CH * NSTEP
NP = 10240
SLC = NP // NS
NPY = N + 16

_MESH = plsc.VectorSubcoreMesh(
    core_axis_name="c", subcore_axis_name="s", num_cores=NC, num_subcores=NS
)


def _sc_gather_scatter(y_pad, src_idx, dst_idx, zeros_blk):
    @functools.partial(
        pl.kernel,
        out_type=jax.ShapeDtypeStruct((NC, NP, H), jnp.float32),
        mesh=_MESH,
        scratch_types=[
            pltpu.VMEM((1, CH), jnp.int32),
            pltpu.VMEM((1, CH), jnp.int32),
            pltpu.VMEM((CH, H), jnp.float32),
            pltpu.VMEM_SHARED((NP, H), jnp.float32),
        ],
    )
    def k(y_hbm, src_hbm, dst_hbm, z_hbm, acc_hbm, srcb, dstb, rows, acc_sh):
        c = lax.axis_index("c")
        s = lax.axis_index("s")
        w = c * NS + s
        pltpu.sync_copy(z_hbm, acc_sh.at[pl.ds(s * SLC, SLC)])
        plsc.subcore_barrier()

        @pl.loop(0, NSTEP)
        def _(i):
            pltpu.sync_copy(src_hbm.at[w, i], srcb)
            pltpu.sync_copy(dst_hbm.at[w, i], dstb)
            pltpu.sync_copy(y_hbm.at[srcb.at[0]], rows)
            pltpu.sync_copy(rows, acc_sh.at[dstb.at[0]], add=True)

        plsc.subcore_barrier()
        pltpu.sync_copy(
            acc_sh.at[pl.ds(s * SLC, SLC)], acc_hbm.at[c, pl.ds(s * SLC, SLC)]
        )

    return k(y_pad, src_idx, dst_idx, zeros_blk)


def _sc_degree(dst_idx, ones_blk, zeros_blk):
    @functools.partial(
        pl.kernel,
        out_type=jax.ShapeDtypeStruct((NC, NP, H), jnp.float32),
        mesh=_MESH,
        scratch_types=[
            pltpu.VMEM((1, CH), jnp.int32),
            pltpu.VMEM((CH, H), jnp.float32),
            pltpu.VMEM_SHARED((NP, H), jnp.float32),
        ],
    )
    def k(dst_hbm, o_hbm, z_hbm, acc_hbm, dstb, rows, acc_sh):
        c = lax.axis_index("c")
        s = lax.axis_index("s")
        w = c * NS + s
        pltpu.sync_copy(o_hbm, rows)
        pltpu.sync_copy(z_hbm, acc_sh.at[pl.ds(s * SLC, SLC)])
        plsc.subcore_barrier()

        @pl.loop(0, NSTEP)
        def _(i):
            pltpu.sync_copy(dst_hbm.at[w, i], dstb)
            pltpu.sync_copy(rows, acc_sh.at[dstb.at[0]], add=True)

        plsc.subcore_barrier()
        pltpu.sync_copy(
            acc_sh.at[pl.ds(s * SLC, SLC)], acc_hbm.at[c, pl.ds(s * SLC, SLC)]
        )

    return k(dst_idx, ones_blk, zeros_blk)


def _tc_first(x, degM, w1t):
    def body(x_ref, deg_ref, w_ref, dinv_ref, y_ref):
        deg = deg_ref[0, :N, :] + deg_ref[1, :N, :] + 1.0
        dinv = lax.rsqrt(deg)
        dinv_ref[...] = dinv
        xw = jnp.dot(x_ref[...], w_ref[...], preferred_element_type=jnp.float32)
        y_ref[:N, :] = xw * dinv
        y_ref[N:, :] = jnp.zeros((NPY - N, H), jnp.float32)

    return pl.pallas_call(
        body,
        out_shape=(
            jax.ShapeDtypeStruct((N, H), jnp.float32),
            jax.ShapeDtypeStruct((NPY, H), jnp.float32),
        ),
    )(x, degM, w1t)


def _bn_relu(z, g_ref, be_ref):
    m = jnp.mean(z, axis=0, keepdims=True)
    zc = z - m
    v = jnp.mean(zc * zc, axis=0, keepdims=True)
    return zc * lax.rsqrt(v + EPS) * g_ref[...] + be_ref[...]


def _tc_mid(acc, y, dinvM, g, be, wnt):
    def body(acc_ref, y_ref, dinv_ref, g_ref, be_ref, w_ref, h_ref, yn_ref):
        z = (acc_ref[0, :N, :] + acc_ref[1, :N, :] + y_ref[:N, :]) * dinv_ref[...]
        h = jnp.maximum(_bn_relu(z, g_ref, be_ref), 0.0)
        h_ref[...] = h
        hw = jnp.dot(h, w_ref[...], preferred_element_type=jnp.float32)
        yn_ref[:N, :] = hw * dinv_ref[...]
        yn_ref[N:, :] = jnp.zeros((NPY - N, H), jnp.float32)

    return pl.pallas_call(
        body,
        out_shape=(
            jax.ShapeDtypeStruct((N, H), jnp.float32),
            jax.ShapeDtypeStruct((NPY, H), jnp.float32),
        ),
    )(acc, y, dinvM, g, be, wnt)


def _tc_res(acc, y, dinvM, g, be, res, wrest, bres, w4t):
    def body(acc_ref, y_ref, dinv_ref, g_ref, be_ref, res_ref, wr_ref, br_ref,
             w_ref, yn_ref):
        z = (acc_ref[0, :N, :] + acc_ref[1, :N, :] + y_ref[:N, :]) * dinv_ref[...]
        bn = _bn_relu(z, g_ref, be_ref)
        rw = jnp.dot(res_ref[...], wr_ref[...], preferred_element_type=jnp.float32)
        h = jnp.maximum(bn + rw + br_ref[...], 0.0)
        hw = jnp.dot(h, w_ref[...], preferred_element_type=jnp.float32)
        yn_ref[:N, :] = hw * dinv_ref[...]
        yn_ref[N:, :] = jnp.zeros((NPY - N, H), jnp.float32)

    return pl.pallas_call(
        body,
        out_shape=jax.ShapeDtypeStruct((NPY, H), jnp.float32),
    )(acc, y, dinvM, g, be, res, wrest, bres, w4t)


def _tc_head(acc, y, dinvM, g, be, wvt, bv, wot, bo, woutt, bout):
    def body(acc_ref, y_ref, dinv_ref, g_ref, be_ref, wv_ref, bv_ref, wo_ref,
             bo_ref, wout_ref, bout_ref, out_ref):
        z = (acc_ref[0, :N, :] + acc_ref[1, :N, :] + y_ref[:N, :]) * dinv_ref[...]
        h = jnp.maximum(_bn_relu(z, g_ref, be_ref), 0.0)
        v = jnp.dot(h, wv_ref[...], preferred_element_type=jnp.float32) + bv_ref[...]
        o = jnp.dot(v, wo_ref[...], preferred_element_type=jnp.float32) + bo_ref[...]
        out_ref[...] = (
            jnp.dot(o, wout_ref[...], preferred_element_type=jnp.float32)
            + bout_ref[...]
        )

    return pl.pallas_call(
        body,
        out_shape=jax.ShapeDtypeStruct((N, H), jnp.float32),
    )(acc, y, dinvM, g, be, wvt, bv, wot, bo, woutt, bout)


def kernel(x, edge_index, params):
    p = params
    pad = jnp.full((EPAD - E,), N, jnp.int32)
    src = jnp.concatenate([edge_index[0], pad]).reshape(NW, NSTEP, 1, CH)
    dst = jnp.concatenate([edge_index[1], pad]).reshape(NW, NSTEP, 1, CH)
    zeros_blk = jnp.zeros((SLC, H), jnp.float32)
    ones_blk = jnp.ones((CH, H), jnp.float32)

    def row(b):
        return b.reshape(1, H)

    degM = _sc_degree(dst, ones_blk, zeros_blk)
    dinvM, y1 = _tc_first(x, degM, p["W1"].T)
    acc1 = _sc_gather_scatter(y1, src, dst, zeros_blk)
    h1, y2 = _tc_mid(acc1, y1, dinvM, row(p["g1"]), row(p["be1"]), p["W2"].T)
    acc2 = _sc_gather_scatter(y2, src, dst, zeros_blk)
    _, y3 = _tc_mid(acc2, y2, dinvM, row(p["g2"]), row(p["be2"]), p["W3"].T)
    acc3 = _sc_gather_scatter(y3, src, dst, zeros_blk)
    y4 = _tc_res(acc3, y3, dinvM, row(p["g3"]), row(p["be3"]), h1,
                 p["Wres"].T, row(p["bres"]), p["W4"].T)
    acc4 = _sc_gather_scatter(y4, src, dst, zeros_blk)
    out = _tc_head(acc4, y4, dinvM, row(p["g4"]), row(p["be4"]),
                   p["Wv"].T, row(p["bv"]), p["Wo"].T, row(p["bo"]),
                   p["Wout"].T, row(p["bout"]))
    return out[None]
